# baseline (speedup 1.0000x reference)
import functools

import jax
import jax.numpy as jnp
from jax.experimental import pallas as pl
from jax.experimental.pallas import tpu as pltpu

B, N, FD = 4, 4096, 64
BN = B * N
_LOG2N = 12


def _shift_right(x, s):
    return jnp.concatenate(
        [jnp.zeros(x.shape[:-1] + (s,), x.dtype), x[..., : x.shape[-1] - s]],
        axis=-1,
    )


def _body(fpass_ref, fT_ref, p3_ref, m_ref, W1f_ref, W1p_ref, b1_ref, W2_ref,
          b2_ref, W3_ref, b3_ref, prob_ref, conf_ref, fout_ref):
    del fpass_ref, fout_ref
    fT = fT_ref[...]
    P3 = p3_ref[...]
    m4 = m_ref[...].astype(jnp.float32)

    pT = P3.reshape(3, BN)
    h1 = jnp.dot(W1f_ref[...], fT, preferred_element_type=jnp.float32)
    h1 += jnp.dot(W1p_ref[...], pT, preferred_element_type=jnp.float32)
    h1 = jnp.maximum(h1 + b1_ref[...], 0.0)
    h2 = jnp.maximum(
        jnp.dot(W2_ref[...], h1, preferred_element_type=jnp.float32)
        + b2_ref[...], 0.0)
    logit = (jnp.dot(W3_ref[...], h2, preferred_element_type=jnp.float32)
             + b3_ref[...])
    srow = jax.nn.sigmoid(logit)
    s4 = jnp.concatenate(
        [srow[:, b * N:(b + 1) * N] for b in range(B)], axis=0)

    cnt = jnp.sum(m4, axis=1, keepdims=True)
    prob = jnp.where(m4 > 0.5, s4, 0.0)
    prob = jnp.where(cnt < 10.0, 0.0, prob)
    prob_ref[...] = prob

    mean = jnp.sum(prob * m4, axis=1, keepdims=True) / jnp.maximum(cnt, 1.0)
    clarity = (jnp.sum(m4 * (prob - mean) ** 2, axis=1, keepdims=True)
               / jnp.maximum(cnt - 1.0, 1.0))

    sel = (prob > 0.7).astype(jnp.float32)
    bcnt = jnp.sum(sel, axis=1, keepdims=True)
    A = jnp.concatenate([P3 * sel[None], sel[None]], axis=0)
    for k in range(_LOG2N):
        A = jnp.where(A[3:4] > 0.5, A, _shift_right(A, 1 << k))
    Ap = _shift_right(A, 1)
    ffprev = Ap[:3]
    hasprev = Ap[3]
    valid = sel * hasprev
    delta = P3 - ffprev
    dsq = jnp.sum(delta * delta, axis=0)
    dist = jnp.sqrt(jnp.maximum(dsq, 1e-24))
    sum_d = jnp.sum(valid * dist, axis=1, keepdims=True)
    pc = jnp.maximum(bcnt - 1.0, 1.0)
    dmean = sum_d / pc
    dvar = (jnp.sum(valid * (dist - dmean) ** 2, axis=1, keepdims=True)
            / jnp.maximum(pc - 1.0, 1.0))
    continuity = jnp.clip(1.0 / (dvar + 1e-8), 0.0, 1.0)
    continuity = jnp.where(bcnt > 5.0, continuity, 0.0)
    conf = jnp.clip(clarity * continuity, 0.0, 1.0)
    conf = jnp.where(cnt == 0.0, 0.0, conf)
    conf_ref[...] = jnp.broadcast_to(conf, (B, 128))


@functools.partial(jax.jit, static_argnames=())
def kernel(points, features, leaf_mask, W1, b1, W2, b2, W3, b3):
    fT = jnp.transpose(features, (2, 0, 1)).reshape(FD, BN)
    P3 = jnp.transpose(points, (2, 0, 1))
    W1f = W1[:, :FD]
    W1p = W1[:, FD:]
    b1c = b1.reshape(FD, 1)
    b2c = b2.reshape(32, 1)
    b3c = b3.reshape(1, 1)

    hbm = pl.BlockSpec(memory_space=pltpu.MemorySpace.HBM)
    vmem = pl.BlockSpec(memory_space=pltpu.MemorySpace.VMEM)
    prob, conf_pad, feats_out = pl.pallas_call(
        _body,
        in_specs=[hbm] + [vmem] * 10,
        out_specs=(vmem, vmem, hbm),
        out_shape=(
            jax.ShapeDtypeStruct((B, N), jnp.float32),
            jax.ShapeDtypeStruct((B, 128), jnp.float32),
            jax.ShapeDtypeStruct((B, N, FD), jnp.float32),
        ),
        input_output_aliases={0: 2},
    )(features, fT, P3, leaf_mask, W1f, W1p, b1c, W2, b2c, W3, b3c)

    return (prob, feats_out, conf_pad[:, 0])


# speedup vs baseline: 1.5661x; 1.5661x over previous
import functools

import jax
import jax.numpy as jnp
from jax.experimental import pallas as pl
from jax.experimental.pallas import tpu as pltpu

B, N, FD = 4, 4096, 64
BN = B * N
_LOG2N = 12


def _shift_right(x, s):
    return jnp.concatenate(
        [jnp.zeros(x.shape[:-1] + (s,), x.dtype), x[..., : x.shape[-1] - s]],
        axis=-1,
    )


def _body(fT_ref, p3_ref, m_ref, W1f_ref, W1p_ref, b1_ref, W2_ref, b2_ref,
          W3_ref, b3_ref, prob_ref, conf_ref):
    fT = fT_ref[...]
    P3 = p3_ref[...]
    m4 = m_ref[...].astype(jnp.float32)

    pT = P3.reshape(3, BN)
    h1 = jnp.dot(W1f_ref[...], fT, preferred_element_type=jnp.float32)
    h1 += jnp.dot(W1p_ref[...], pT, preferred_element_type=jnp.float32)
    h1 = jnp.maximum(h1 + b1_ref[...], 0.0)
    h2 = jnp.maximum(
        jnp.dot(W2_ref[...], h1, preferred_element_type=jnp.float32)
        + b2_ref[...], 0.0)
    logit = (jnp.dot(W3_ref[...], h2, preferred_element_type=jnp.float32)
             + b3_ref[...])
    srow = jax.nn.sigmoid(logit)
    s4 = jnp.concatenate(
        [srow[:, b * N:(b + 1) * N] for b in range(B)], axis=0)

    cnt = jnp.sum(m4, axis=1, keepdims=True)
    prob = jnp.where(m4 > 0.5, s4, 0.0)
    prob = jnp.where(cnt < 10.0, 0.0, prob)
    prob_ref[...] = prob

    mean = jnp.sum(prob * m4, axis=1, keepdims=True) / jnp.maximum(cnt, 1.0)
    clarity = (jnp.sum(m4 * (prob - mean) ** 2, axis=1, keepdims=True)
               / jnp.maximum(cnt - 1.0, 1.0))

    sel = (prob > 0.7).astype(jnp.float32)
    bcnt = jnp.sum(sel, axis=1, keepdims=True)
    A = jnp.concatenate([P3 * sel[None], sel[None]], axis=0)
    for k in range(_LOG2N):
        A = jnp.where(A[3:4] > 0.5, A, _shift_right(A, 1 << k))
    Ap = _shift_right(A, 1)
    ffprev = Ap[:3]
    hasprev = Ap[3]
    valid = sel * hasprev
    delta = P3 - ffprev
    dsq = jnp.sum(delta * delta, axis=0)
    dist = jnp.sqrt(jnp.maximum(dsq, 1e-24))
    sum_d = jnp.sum(valid * dist, axis=1, keepdims=True)
    pc = jnp.maximum(bcnt - 1.0, 1.0)
    dmean = sum_d / pc
    dvar = (jnp.sum(valid * (dist - dmean) ** 2, axis=1, keepdims=True)
            / jnp.maximum(pc - 1.0, 1.0))
    continuity = jnp.clip(1.0 / (dvar + 1e-8), 0.0, 1.0)
    continuity = jnp.where(bcnt > 5.0, continuity, 0.0)
    conf = jnp.clip(clarity * continuity, 0.0, 1.0)
    conf = jnp.where(cnt == 0.0, 0.0, conf)
    conf_ref[...] = jnp.concatenate(
        [conf[b:b + 1, :] for b in range(B)], axis=1)


@functools.partial(jax.jit, static_argnames=())
def kernel(points, features, leaf_mask, W1, b1, W2, b2, W3, b3):
    fT = jnp.transpose(features, (2, 0, 1)).reshape(FD, BN)
    P3 = jnp.transpose(points, (2, 0, 1))
    W1f = W1[:, :FD]
    W1p = W1[:, FD:]
    b1c = b1.reshape(FD, 1)
    b2c = b2.reshape(32, 1)
    b3c = b3.reshape(1, 1)

    prob, conf_row = pl.pallas_call(
        _body,
        out_shape=(
            jax.ShapeDtypeStruct((B, N), jnp.float32),
            jax.ShapeDtypeStruct((1, B), jnp.float32),
        ),
    )(fT, P3, leaf_mask, W1f, W1p, b1c, W2, b2c, W3, b3c)

    return (prob, features, conf_row.reshape(B))
